# baseline (speedup 1.0000x reference)
import functools

import jax
import jax.numpy as jnp
from jax import lax
from jax.experimental import pallas as pl
from jax.experimental.pallas import tpu as pltpu
from jax.experimental.pallas import tpu_sc as plsc

S = 2048
H = 768
FF = 1536
E = 8
V = 16000
TM = 256
NROWS = 4096 + E * TM
NTILES = NROWS // TM
NW = 32
TPW = S // NW

_f32 = jnp.float32
_i32 = jnp.int32
_MM = jnp.bfloat16


def _sc_mesh():
    return plsc.VectorSubcoreMesh(core_axis_name="c", subcore_axis_name="s")


def _wid():
    return lax.axis_index("s") * 2 + lax.axis_index("c")


def _emb_gather(table, idx):
    return jnp.take(table, idx, axis=0)

    @functools.partial(
        pl.kernel,
        mesh=_sc_mesh(),
        out_type=jax.ShapeDtypeStruct((S, H), _f32),
        scratch_types=[
            pltpu.VMEM((TPW,), _i32),
            pltpu.VMEM((TPW, H), _f32),
            pltpu.SemaphoreType.DMA,
        ],
    )
    def k(table_hbm, idx_hbm, out_hbm, idx_v, rows_v, sem):
        base = _wid() * TPW
        pltpu.sync_copy(idx_hbm.at[pl.ds(base, TPW)], idx_v)
        pltpu.async_copy(table_hbm.at[idx_v], rows_v, sem).wait()
        pltpu.sync_copy(rows_v, out_hbm.at[pl.ds(base, TPW)])

    return k(table, idx)


def _row_scatter2(h, p1, p2):
    rows = jnp.zeros((NROWS, H), _f32)
    return rows.at[p1].set(h).at[p2].set(h)

    @functools.partial(
        pl.kernel,
        mesh=_sc_mesh(),
        out_type=jax.ShapeDtypeStruct((NROWS, H), _f32),
        scratch_types=[
            pltpu.VMEM((TPW,), _i32),
            pltpu.VMEM((TPW,), _i32),
            pltpu.VMEM((TPW, H), _f32),
            pltpu.SemaphoreType.DMA,
        ],
    )
    def k(h_hbm, p1_hbm, p2_hbm, rows_hbm, i1_v, i2_v, buf, sem):
        base = _wid() * TPW
        pltpu.sync_copy(p1_hbm.at[pl.ds(base, TPW)], i1_v)
        pltpu.sync_copy(p2_hbm.at[pl.ds(base, TPW)], i2_v)
        pltpu.sync_copy(h_hbm.at[pl.ds(base, TPW)], buf)
        c1 = pltpu.async_copy(buf, rows_hbm.at[i1_v], sem)
        c2 = pltpu.async_copy(buf, rows_hbm.at[i2_v], sem)
        c1.wait()
        c2.wait()

    return k(h, p1, p2)


def _row_gather2(y, p1, p2):
    return jnp.take(y, p1, axis=0), jnp.take(y, p2, axis=0)

    @functools.partial(
        pl.kernel,
        mesh=_sc_mesh(),
        out_type=(
            jax.ShapeDtypeStruct((S, H), _f32),
            jax.ShapeDtypeStruct((S, H), _f32),
        ),
        scratch_types=[
            pltpu.VMEM((TPW,), _i32),
            pltpu.VMEM((TPW,), _i32),
            pltpu.VMEM((TPW, H), _f32),
            pltpu.VMEM((TPW, H), _f32),
            pltpu.SemaphoreType.DMA,
        ],
    )
    def k(y_hbm, p1_hbm, p2_hbm, o1_hbm, o2_hbm, i1_v, i2_v, b1_v, b2_v, sem):
        base = _wid() * TPW
        pltpu.sync_copy(p1_hbm.at[pl.ds(base, TPW)], i1_v)
        pltpu.sync_copy(p2_hbm.at[pl.ds(base, TPW)], i2_v)
        c1 = pltpu.async_copy(y_hbm.at[i1_v], b1_v, sem)
        c2 = pltpu.async_copy(y_hbm.at[i2_v], b2_v, sem)
        c1.wait()
        c2.wait()
        pltpu.sync_copy(b1_v, o1_hbm.at[pl.ds(base, TPW)])
        pltpu.sync_copy(b2_v, o2_hbm.at[pl.ds(base, TPW)])

    return k(y, p1, p2)


def _routing_body(has_combine, refs):
    if has_combine:
        (x_ref, y1_ref, y2_ref, gin_ref, g_ref, b_ref, wr_ref,
         xo_ref, h_ref, p1_ref, p2_ref, gates_ref, ept_ref) = refs
        gin = gin_ref[...]
        x = (x_ref[...]
             + gin[:, 0:1] * y1_ref[...]
             + gin[:, 1:2] * y2_ref[...])
    else:
        (x_ref, add_ref, g_ref, b_ref, wr_ref,
         xo_ref, h_ref, p1_ref, p2_ref, gates_ref, ept_ref) = refs
        x = x_ref[...] + add_ref[...]
    xo_ref[...] = x

    m = jnp.mean(x, axis=-1, keepdims=True)
    d = x - m
    v = jnp.mean(d * d, axis=-1, keepdims=True)
    h = d * lax.rsqrt(v + 1e-5) * g_ref[...] + b_ref[...]
    h_ref[...] = h

    logits = jnp.dot(h, wr_ref[...], preferred_element_type=_f32)
    z = logits - jnp.max(logits, axis=-1, keepdims=True)
    ez = jnp.exp(z)
    p = ez / jnp.sum(ez, axis=-1, keepdims=True)

    io8 = lax.broadcasted_iota(_i32, (S, E), 1)
    m1 = jnp.max(p, axis=-1, keepdims=True)
    idx1 = jnp.min(jnp.where(p >= m1, io8, 8), axis=-1, keepdims=True)
    pm = jnp.where(io8 == idx1, -1.0, p)
    m2 = jnp.max(pm, axis=-1, keepdims=True)
    idx2 = jnp.min(jnp.where(pm >= m2, io8, 8), axis=-1, keepdims=True)

    denom = m1 + m2 + 1e-9
    gates_ref[...] = jnp.concatenate([m1 / denom, m2 / denom], axis=-1)

    oh1 = (io8 == idx1).astype(_f32)
    oh2 = (io8 == idx2).astype(_f32)
    cnt2 = oh1 + oh2

    c = cnt2
    sshift = 1
    while sshift < S:
        c = c + jnp.concatenate(
            [jnp.zeros((sshift, E), _f32), c[: S - sshift]], axis=0)
        sshift *= 2
    cex = c - cnt2

    counts = jnp.sum(cnt2, axis=0, keepdims=True)
    pc = jnp.floor((counts + (TM - 1)) / TM) * TM
    iu = lax.broadcasted_iota(_i32, (E, E), 0)
    ju = lax.broadcasted_iota(_i32, (E, E), 1)
    upper = (iu < ju).astype(_f32)
    start = jnp.dot(pc, upper, preferred_element_type=_f32)

    r1 = jnp.sum(oh1 * cex, axis=-1, keepdims=True)
    r2 = jnp.sum(oh2 * cex, axis=-1, keepdims=True)
    s1 = jnp.sum(oh1 * start, axis=-1, keepdims=True)
    s2 = jnp.sum(oh2 * start, axis=-1, keepdims=True)
    p1_ref[...] = (r1 + s1).astype(_i32)
    p2_ref[...] = (r2 + s2).astype(_i32)

    tau = (lax.broadcasted_iota(_i32, (NTILES + 8, E), 0) * TM).astype(_f32)
    ept = jnp.sum((start <= tau).astype(_f32), axis=-1, keepdims=True) - 1.0
    ept_ref[...] = jnp.clip(ept, 0.0, E - 1).astype(_i32)


def _routing_call(x, y1_or_add, y2, gin, ln_g, ln_b, wr):
    has_combine = y2 is not None
    body = functools.partial(_routing_body, has_combine)

    def wrapped(*refs):
        body(refs)

    out_shape = (
        jax.ShapeDtypeStruct((S, H), _f32),
        jax.ShapeDtypeStruct((S, H), _f32),
        jax.ShapeDtypeStruct((S, 1), _i32),
        jax.ShapeDtypeStruct((S, 1), _i32),
        jax.ShapeDtypeStruct((S, 2), _f32),
        jax.ShapeDtypeStruct((NTILES + 8, 1), _i32),
    )
    if has_combine:
        ins = (x, y1_or_add, y2, gin, ln_g, ln_b, wr)
    else:
        ins = (x, y1_or_add, ln_g, ln_b, wr)
    return pl.pallas_call(wrapped, out_shape=out_shape)(*ins)


def _ffn_body(mm, ept_ref, rows_ref, w1_ref, b1_ref, w2_ref, b2_ref, y_ref):
    a = rows_ref[...].astype(mm)
    z = jnp.dot(a, w1_ref[0].astype(mm),
                preferred_element_type=_f32) + b1_ref[0]
    z = jax.nn.gelu(z)
    y = jnp.dot(z.astype(mm), w2_ref[0].astype(mm),
                preferred_element_type=_f32) + b2_ref[0]
    y_ref[...] = y


def _ffn_call(ept, rows, w1, b1, w2, b2, mm):
    grid_spec = pltpu.PrefetchScalarGridSpec(
        num_scalar_prefetch=1,
        grid=(NTILES,),
        in_specs=[
            pl.BlockSpec((TM, H), lambda t, ept: (t, 0)),
            pl.BlockSpec((1, H, FF), lambda t, ept: (ept[t], 0, 0)),
            pl.BlockSpec((1, 1, FF), lambda t, ept: (ept[t], 0, 0)),
            pl.BlockSpec((1, FF, H), lambda t, ept: (ept[t], 0, 0)),
            pl.BlockSpec((1, 1, H), lambda t, ept: (ept[t], 0, 0)),
        ],
        out_specs=pl.BlockSpec((TM, H), lambda t, ept: (t, 0)),
    )
    return pl.pallas_call(
        functools.partial(_ffn_body, mm),
        grid_spec=grid_spec,
        out_shape=jax.ShapeDtypeStruct((NROWS, H), _f32),
    )(ept, rows, w1, b1, w2, b2)


_FM = 512
_FN = 3200


def _final_body(x_ref, y1_ref, y2_ref, gin_ref, lg_ref, lb_ref,
                wout_ref, bout_ref, out_ref):
    gin = gin_ref[...]
    x = (x_ref[...]
         + gin[:, 0:1] * y1_ref[...]
         + gin[:, 1:2] * y2_ref[...])
    m = jnp.mean(x, axis=-1, keepdims=True)
    d = x - m
    v = jnp.mean(d * d, axis=-1, keepdims=True)
    h = d * lax.rsqrt(v + 1e-5) * lg_ref[...] + lb_ref[...]
    out_ref[...] = jnp.dot(
        h.astype(_MM), wout_ref[...].astype(_MM),
        preferred_element_type=_f32) + bout_ref[...]


def _final_call(x, y1, y2, gin, lnf_g, lnf_b, wout, bout):
    return pl.pallas_call(
        _final_body,
        grid=(S // _FM, V // _FN),
        in_specs=[
            pl.BlockSpec((_FM, H), lambda i, j: (i, 0)),
            pl.BlockSpec((_FM, H), lambda i, j: (i, 0)),
            pl.BlockSpec((_FM, H), lambda i, j: (i, 0)),
            pl.BlockSpec((_FM, 2), lambda i, j: (i, 0)),
            pl.BlockSpec((1, H), lambda i, j: (0, 0)),
            pl.BlockSpec((1, H), lambda i, j: (0, 0)),
            pl.BlockSpec((H, _FN), lambda i, j: (0, j)),
            pl.BlockSpec((1, _FN), lambda i, j: (0, j)),
        ],
        out_specs=pl.BlockSpec((_FM, _FN), lambda i, j: (i, j)),
        out_shape=jax.ShapeDtypeStruct((S, V), _f32),
    )(x, y1, y2, gin, lnf_g, lnf_b, wout, bout)


def kernel(input_ids, token_embed, pos_embed, ln_g, ln_b, Wr, W1, b1, W2, b2,
           lnf_g, lnf_b, Wout, bout):
    ids = input_ids.reshape(S)
    x0 = _emb_gather(token_embed, ids)

    x, y1, y2, gates = x0, None, None, None
    add = pos_embed[:S]
    num_layers = Wr.shape[0]
    for l in range(num_layers):
        if l == 0:
            outs = _routing_call(x, add, None, None,
                                 ln_g[l].reshape(1, H), ln_b[l].reshape(1, H),
                                 Wr[l])
        else:
            outs = _routing_call(x, y1, y2, gates,
                                 ln_g[l].reshape(1, H), ln_b[l].reshape(1, H),
                                 Wr[l])
        x, h, p1, p2, gates, ept = outs
        p1 = p1.reshape(S)
        p2 = p2.reshape(S)
        ept = ept.reshape(NTILES + 8)[:NTILES]
        rows = _row_scatter2(h, p1, p2)
        mm = _f32 if l + 1 < num_layers else _MM
        y = _ffn_call(ept, rows, W1[l], b1[l].reshape(E, 1, FF),
                      W2[l], b2[l].reshape(E, 1, H), mm)
        y1, y2 = _row_gather2(y, p1, p2)

    logits = _final_call(x, y1, y2, gates,
                         lnf_g.reshape(1, H), lnf_b.reshape(1, H),
                         Wout, bout.reshape(1, V))
    return logits.reshape(1, S, V)


# speedup vs baseline: 1.5319x; 1.5319x over previous
import functools

import jax
import jax.numpy as jnp
from jax import lax
from jax.experimental import pallas as pl
from jax.experimental.pallas import tpu as pltpu
from jax.experimental.pallas import tpu_sc as plsc

S = 2048
H = 768
FF = 1536
E = 8
V = 16000
TM = 256
NROWS = 4096 + E * TM
NTILES = NROWS // TM
NW = 32
TPW = S // NW

_f32 = jnp.float32
_i32 = jnp.int32
_MM = jnp.bfloat16


def _sc_mesh():
    return plsc.VectorSubcoreMesh(core_axis_name="c", subcore_axis_name="s")


def _wid():
    return lax.axis_index("s") * 2 + lax.axis_index("c")


def _emb_gather(table, idx):
    @functools.partial(
        pl.kernel,
        mesh=_sc_mesh(),
        out_type=jax.ShapeDtypeStruct((S, H), _f32),
        scratch_types=[
            pltpu.VMEM((TPW,), _i32),
            pltpu.VMEM((TPW, H), _f32),
            pltpu.SemaphoreType.DMA,
        ],
    )
    def k(table_hbm, idx_hbm, out_hbm, idx_v, rows_v, sem):
        base = _wid() * TPW
        pltpu.sync_copy(idx_hbm.at[pl.ds(base, TPW)], idx_v)
        pltpu.async_copy(table_hbm.at[idx_v], rows_v, sem).wait()
        pltpu.sync_copy(rows_v, out_hbm.at[pl.ds(base, TPW)])

    return k(table, idx)


def _row_scatter2(h, p1, p2):
    @functools.partial(
        pl.kernel,
        mesh=_sc_mesh(),
        out_type=jax.ShapeDtypeStruct((NROWS, H), _f32),
        scratch_types=[
            pltpu.VMEM((TPW,), _i32),
            pltpu.VMEM((TPW,), _i32),
            pltpu.VMEM((TPW, H), _f32),
            pltpu.SemaphoreType.DMA,
        ],
    )
    def k(h_hbm, p1_hbm, p2_hbm, rows_hbm, i1_v, i2_v, buf, sem):
        base = _wid() * TPW
        pltpu.sync_copy(p1_hbm.at[pl.ds(base, TPW)], i1_v)
        pltpu.sync_copy(p2_hbm.at[pl.ds(base, TPW)], i2_v)
        pltpu.sync_copy(h_hbm.at[pl.ds(base, TPW)], buf)
        c1 = pltpu.async_copy(buf, rows_hbm.at[i1_v], sem)
        c2 = pltpu.async_copy(buf, rows_hbm.at[i2_v], sem)
        c1.wait()
        c2.wait()

    return k(h, p1, p2)


def _row_gather2(y, p1, p2):
    @functools.partial(
        pl.kernel,
        mesh=_sc_mesh(),
        out_type=(
            jax.ShapeDtypeStruct((S, H), _f32),
            jax.ShapeDtypeStruct((S, H), _f32),
        ),
        scratch_types=[
            pltpu.VMEM((TPW,), _i32),
            pltpu.VMEM((TPW,), _i32),
            pltpu.VMEM((TPW, H), _f32),
            pltpu.VMEM((TPW, H), _f32),
            pltpu.SemaphoreType.DMA,
        ],
    )
    def k(y_hbm, p1_hbm, p2_hbm, o1_hbm, o2_hbm, i1_v, i2_v, b1_v, b2_v, sem):
        base = _wid() * TPW
        pltpu.sync_copy(p1_hbm.at[pl.ds(base, TPW)], i1_v)
        pltpu.sync_copy(p2_hbm.at[pl.ds(base, TPW)], i2_v)
        c1 = pltpu.async_copy(y_hbm.at[i1_v], b1_v, sem)
        c2 = pltpu.async_copy(y_hbm.at[i2_v], b2_v, sem)
        c1.wait()
        c2.wait()
        pltpu.sync_copy(b1_v, o1_hbm.at[pl.ds(base, TPW)])
        pltpu.sync_copy(b2_v, o2_hbm.at[pl.ds(base, TPW)])

    return k(y, p1, p2)


def _routing_body(has_combine, refs):
    if has_combine:
        (x_ref, y1_ref, y2_ref, gin_ref, g_ref, b_ref, wr_ref,
         xo_ref, h_ref, p1_ref, p2_ref, gates_ref, ept_ref) = refs
        gin = gin_ref[...]
        x = (x_ref[...]
             + gin[:, 0:1] * y1_ref[...]
             + gin[:, 1:2] * y2_ref[...])
    else:
        (x_ref, add_ref, g_ref, b_ref, wr_ref,
         xo_ref, h_ref, p1_ref, p2_ref, gates_ref, ept_ref) = refs
        x = x_ref[...] + add_ref[...]
    xo_ref[...] = x

    m = jnp.mean(x, axis=-1, keepdims=True)
    d = x - m
    v = jnp.mean(d * d, axis=-1, keepdims=True)
    h = d * lax.rsqrt(v + 1e-5) * g_ref[...] + b_ref[...]
    h_ref[...] = h

    logits = jnp.dot(h, wr_ref[...], preferred_element_type=_f32)
    z = logits - jnp.max(logits, axis=-1, keepdims=True)
    ez = jnp.exp(z)
    p = ez / jnp.sum(ez, axis=-1, keepdims=True)

    io8 = lax.broadcasted_iota(_i32, (S, E), 1)
    m1 = jnp.max(p, axis=-1, keepdims=True)
    idx1 = jnp.min(jnp.where(p >= m1, io8, 8), axis=-1, keepdims=True)
    pm = jnp.where(io8 == idx1, -1.0, p)
    m2 = jnp.max(pm, axis=-1, keepdims=True)
    idx2 = jnp.min(jnp.where(pm >= m2, io8, 8), axis=-1, keepdims=True)

    denom = m1 + m2 + 1e-9
    gates_ref[...] = jnp.concatenate([m1 / denom, m2 / denom], axis=-1)

    oh1 = (io8 == idx1).astype(_f32)
    oh2 = (io8 == idx2).astype(_f32)
    cnt2 = oh1 + oh2

    c = cnt2
    sshift = 1
    while sshift < S:
        c = c + jnp.concatenate(
            [jnp.zeros((sshift, E), _f32), c[: S - sshift]], axis=0)
        sshift *= 2
    cex = c - cnt2

    counts = jnp.sum(cnt2, axis=0, keepdims=True)
    pc = jnp.floor((counts + (TM - 1)) / TM) * TM
    iu = lax.broadcasted_iota(_i32, (E, E), 0)
    ju = lax.broadcasted_iota(_i32, (E, E), 1)
    upper = (iu < ju).astype(_f32)
    start = jnp.dot(pc, upper, preferred_element_type=_f32)

    r1 = jnp.sum(oh1 * cex, axis=-1, keepdims=True)
    r2 = jnp.sum(oh2 * cex, axis=-1, keepdims=True)
    s1 = jnp.sum(oh1 * start, axis=-1, keepdims=True)
    s2 = jnp.sum(oh2 * start, axis=-1, keepdims=True)
    p1_ref[...] = (r1 + s1).astype(_i32)
    p2_ref[...] = (r2 + s2).astype(_i32)

    tau = (lax.broadcasted_iota(_i32, (NTILES + 8, E), 0) * TM).astype(_f32)
    ept = jnp.sum((start <= tau).astype(_f32), axis=-1, keepdims=True) - 1.0
    ept_ref[...] = jnp.clip(ept, 0.0, E - 1).astype(_i32)


def _routing_call(x, y1_or_add, y2, gin, ln_g, ln_b, wr):
    has_combine = y2 is not None
    body = functools.partial(_routing_body, has_combine)

    def wrapped(*refs):
        body(refs)

    out_shape = (
        jax.ShapeDtypeStruct((S, H), _f32),
        jax.ShapeDtypeStruct((S, H), _f32),
        jax.ShapeDtypeStruct((S, 1), _i32),
        jax.ShapeDtypeStruct((S, 1), _i32),
        jax.ShapeDtypeStruct((S, 2), _f32),
        jax.ShapeDtypeStruct((NTILES + 8, 1), _i32),
    )
    if has_combine:
        ins = (x, y1_or_add, y2, gin, ln_g, ln_b, wr)
    else:
        ins = (x, y1_or_add, ln_g, ln_b, wr)
    return pl.pallas_call(wrapped, out_shape=out_shape)(*ins)


def _ffn_body(mm, ept_ref, rows_ref, w1_ref, b1_ref, w2_ref, b2_ref, y_ref):
    a = rows_ref[...].astype(mm)
    z = jnp.dot(a, w1_ref[0, 0].astype(mm),
                preferred_element_type=_f32) + b1_ref[0, 0]
    z = jax.nn.gelu(z)
    y = jnp.dot(z.astype(mm), w2_ref[0, 0].astype(mm),
                preferred_element_type=_f32) + b2_ref[0, 0]
    y_ref[...] = y


def _ffn_call(l, ept, rows, w1, b1, w2, b2, mm):
    grid_spec = pltpu.PrefetchScalarGridSpec(
        num_scalar_prefetch=1,
        grid=(NTILES,),
        in_specs=[
            pl.BlockSpec((TM, H), lambda t, ept: (t, 0)),
            pl.BlockSpec((1, 1, H, FF), lambda t, ept: (l, ept[t], 0, 0)),
            pl.BlockSpec((1, 1, 1, FF), lambda t, ept: (l, ept[t], 0, 0)),
            pl.BlockSpec((1, 1, FF, H), lambda t, ept: (l, ept[t], 0, 0)),
            pl.BlockSpec((1, 1, 1, H), lambda t, ept: (l, ept[t], 0, 0)),
        ],
        out_specs=pl.BlockSpec((TM, H), lambda t, ept: (t, 0)),
    )
    return pl.pallas_call(
        functools.partial(_ffn_body, mm),
        grid_spec=grid_spec,
        out_shape=jax.ShapeDtypeStruct((NROWS, H), _f32),
    )(ept, rows, w1, b1, w2, b2)


_FM = 512
_FN = 3200


def _final_body(x_ref, y1_ref, y2_ref, gin_ref, lg_ref, lb_ref,
                wout_ref, bout_ref, out_ref):
    gin = gin_ref[...]
    x = (x_ref[...]
         + gin[:, 0:1] * y1_ref[...]
         + gin[:, 1:2] * y2_ref[...])
    m = jnp.mean(x, axis=-1, keepdims=True)
    d = x - m
    v = jnp.mean(d * d, axis=-1, keepdims=True)
    h = d * lax.rsqrt(v + 1e-5) * lg_ref[...] + lb_ref[...]
    out_ref[...] = jnp.dot(
        h.astype(_MM), wout_ref[...].astype(_MM),
        preferred_element_type=_f32) + bout_ref[...]


def _final_call(x, y1, y2, gin, lnf_g, lnf_b, wout, bout):
    return pl.pallas_call(
        _final_body,
        grid=(S // _FM, V // _FN),
        in_specs=[
            pl.BlockSpec((_FM, H), lambda i, j: (i, 0)),
            pl.BlockSpec((_FM, H), lambda i, j: (i, 0)),
            pl.BlockSpec((_FM, H), lambda i, j: (i, 0)),
            pl.BlockSpec((_FM, 2), lambda i, j: (i, 0)),
            pl.BlockSpec((1, H), lambda i, j: (0, 0)),
            pl.BlockSpec((1, H), lambda i, j: (0, 0)),
            pl.BlockSpec((H, _FN), lambda i, j: (0, j)),
            pl.BlockSpec((1, _FN), lambda i, j: (0, j)),
        ],
        out_specs=pl.BlockSpec((_FM, _FN), lambda i, j: (i, j)),
        out_shape=jax.ShapeDtypeStruct((S, V), _f32),
    )(x, y1, y2, gin, lnf_g, lnf_b, wout, bout)


def kernel(input_ids, token_embed, pos_embed, ln_g, ln_b, Wr, W1, b1, W2, b2,
           lnf_g, lnf_b, Wout, bout):
    ids = input_ids.reshape(S)
    x0 = _emb_gather(token_embed, ids)

    x, y1, y2, gates = x0, None, None, None
    add = pos_embed[:S]
    num_layers = Wr.shape[0]
    for l in range(num_layers):
        if l == 0:
            outs = _routing_call(x, add, None, None,
                                 ln_g[l].reshape(1, H), ln_b[l].reshape(1, H),
                                 Wr[l])
        else:
            outs = _routing_call(x, y1, y2, gates,
                                 ln_g[l].reshape(1, H), ln_b[l].reshape(1, H),
                                 Wr[l])
        x, h, p1, p2, gates, ept = outs
        p1 = p1.reshape(S)
        p2 = p2.reshape(S)
        ept = ept.reshape(NTILES + 8)[:NTILES]
        rows = _row_scatter2(h, p1, p2)
        mm = _f32 if l + 1 < num_layers else _MM
        y = _ffn_call(l, ept, rows, W1,
                      b1.reshape(num_layers, E, 1, FF),
                      W2, b2.reshape(num_layers, E, 1, H), mm)
        y1, y2 = _row_gather2(y, p1, p2)

    logits = _final_call(x, y1, y2, gates,
                         lnf_g.reshape(1, H), lnf_b.reshape(1, H),
                         Wout, bout.reshape(1, V))
    return logits.reshape(1, S, V)
